# Initial kernel scaffold; baseline (speedup 1.0000x reference)
import jax
import jax.numpy as jnp
from jax.experimental import pallas as pl


def kernel(x, edge_index, edge_attr, g1_lin, g1_edge, g1_att, g2_lin, g2_edge, g2_att, g3_lin, g3_edge, g3_att, W1, b1, W2, b2, W3, b3, Ws1, bs1, Ws2, bs2, Ws3, bs3, bn1_g, bn1_b, bn2_g, bn2_b, ln_g, ln_b):
    raise NotImplementedError("write your pallas kernel here")


# speedup vs baseline: 42.0740x; 42.0740x over previous
import functools
import jax
import jax.numpy as jnp
from jax import lax
from jax.experimental import pallas as pl
from jax.experimental.pallas import tpu as pltpu
from jax.experimental.pallas import tpu_sc as plsc

N = 10000
E = 320000
D = 128
H = 8
OC = 16

NC = 2
NS = 16
NW = NC * NS
EPW = E // NW
K = 80
NCHUNK = EPW // K
ROWS_PT = 624

f32 = jnp.float32


def _bcast_lane(v, k):
    idx = jnp.full((16, 1), k, jnp.int32)
    return lax.gather(
        v, idx,
        dimension_numbers=lax.GatherDimensionNumbers(
            offset_dims=(), collapsed_slice_dims=(0,), start_index_map=(0,)),
        slice_sizes=(1,),
        mode=lax.GatherScatterMode.PROMISE_IN_BOUNDS)


def _sc_gat(row, col, ea, lin, sij, ce16, multi_head):
    mesh = plsc.VectorSubcoreMesh(
        core_axis_name="c", subcore_axis_name="s", num_cores=NC, num_subcores=NS)

    @functools.partial(
        pl.kernel,
        out_type=(jax.ShapeDtypeStruct((NC, N, D), f32),
                  jax.ShapeDtypeStruct((NC, N, 16), f32)),
        mesh=mesh,
        compiler_params=pltpu.CompilerParams(needs_layout_passes=False,
                                             use_tc_tiling_on_sc=False),
        scratch_types=[
            pltpu.VMEM((K,), jnp.int32),
            pltpu.VMEM((K,), jnp.int32),
            pltpu.VMEM((K,), f32),
            pltpu.VMEM((K, 16), f32),
            pltpu.VMEM((K, 16), f32),
            pltpu.VMEM((K, 16), f32),
            pltpu.VMEM((K, D), f32),
            pltpu.VMEM((16,), f32),
            pltpu.VMEM_SHARED((N, D), f32),
            pltpu.VMEM_SHARED((N, 16), f32),
            pltpu.SemaphoreType.DMA,
        ],
    )
    def gat_kernel(row_h, col_h, ea_h, lin_h, sij_h, ce_h, num_h, den_h,
                   idxr, idxc, eav, sr, scb, exb, linb, cev, shacc, shden, sem):
        c = lax.axis_index("c")
        s = lax.axis_index("s")
        wid = c * NS + s
        io = lax.iota(jnp.int32, 16)
        iob = io >> 3
        io7 = io & 7
        zf = jnp.zeros((16,), f32)

        @pl.loop(0, K)
        def _zero(r):
            for kk in range(D // 16):
                linb[r, pl.ds(kk * 16, 16)] = zf
            exb[r, :] = zf

        rb = pl.multiple_of(s * ROWS_PT, 8)
        for t in range(ROWS_PT // K):
            pltpu.sync_copy(linb, shacc.at[pl.ds(rb + t * K, K)])
            pltpu.sync_copy(exb, shden.at[pl.ds(rb + t * K, K)])
        rem = ROWS_PT % K
        if rem:
            pltpu.sync_copy(linb.at[pl.ds(0, rem)],
                            shacc.at[pl.ds(rb + (ROWS_PT // K) * K, rem)])
            pltpu.sync_copy(exb.at[pl.ds(0, rem)],
                            shden.at[pl.ds(rb + (ROWS_PT // K) * K, rem)])
        tail = N - NS * ROWS_PT

        @pl.when(s == NS - 1)
        def _zero_tail():
            pltpu.sync_copy(linb.at[pl.ds(0, tail)],
                            shacc.at[pl.ds(NS * ROWS_PT, tail)])
            pltpu.sync_copy(exb.at[pl.ds(0, tail)],
                            shden.at[pl.ds(NS * ROWS_PT, tail)])

        plsc.subcore_barrier()

        pltpu.sync_copy(ce_h, cev)
        cv = cev[:]

        ebase = wid * EPW

        @pl.loop(0, NCHUNK)
        def _chunk(q):
            base = pl.multiple_of(ebase + q * K, 8)
            pltpu.sync_copy(row_h.at[pl.ds(base, K)], idxr)
            pltpu.sync_copy(col_h.at[pl.ds(base, K)], idxc)
            pltpu.sync_copy(ea_h.at[pl.ds(base, K)], eav)
            pltpu.async_copy(sij_h.at[idxr], sr, sem).wait()
            pltpu.async_copy(sij_h.at[idxc], scb, sem).wait()
            pltpu.async_copy(lin_h.at[idxc], linb, sem).wait()

            if multi_head:
                @pl.loop(0, K // 2)
                def _pair(i):
                    e2 = i * 2
                    rsel = e2 + iob
                    si_v = plsc.load_gather(sr, [rsel, io7])
                    sj_v = plsc.load_gather(scb, [rsel, io7 + 8])
                    ea_v = plsc.load_gather(eav, [rsel])
                    al = si_v + sj_v + ea_v * cv
                    al = jnp.maximum(al, 0.2 * al)
                    ex = jnp.exp(al)
                    plsc.store_scatter(exb, [rsel, io7], ex)
                    for kk in range(H):
                        sl = pl.ds(kk * 16, 16)
                        linb[e2, sl] = linb[e2, sl] * _bcast_lane(ex, kk)
                        linb[e2 + 1, sl] = linb[e2 + 1, sl] * _bcast_lane(ex, 8 + kk)
            else:
                io0 = io * 0

                @pl.loop(0, K // 16)
                def _grp(j):
                    e0 = j * 16
                    rows = e0 + io
                    si_v = plsc.load_gather(sr, [rows, io0])
                    sj_v = plsc.load_gather(scb, [rows, io0 + 1])
                    ea_v = eav[pl.ds(e0, 16)]
                    al = si_v + sj_v + ea_v * cv
                    al = jnp.maximum(al, 0.2 * al)
                    ex = jnp.exp(al)
                    plsc.store_scatter(exb, [rows, io0], ex)
                    for l in range(16):
                        m = _bcast_lane(ex, l)
                        for kk in range(D // 16):
                            sl = pl.ds(kk * 16, 16)
                            linb[e0 + l, sl] = linb[e0 + l, sl] * m

            pltpu.sync_copy(exb, shden.at[idxr], add=True)
            pltpu.sync_copy(linb, shacc.at[idxr], add=True)

        plsc.subcore_barrier()
        pltpu.sync_copy(shacc.at[pl.ds(rb, ROWS_PT)],
                        num_h.at[c].at[pl.ds(rb, ROWS_PT)])
        pltpu.sync_copy(shden.at[pl.ds(rb, ROWS_PT)],
                        den_h.at[c].at[pl.ds(rb, ROWS_PT)])

        @pl.when(s == NS - 1)
        def _copy_tail():
            pltpu.sync_copy(shacc.at[pl.ds(NS * ROWS_PT, tail)],
                            num_h.at[c].at[pl.ds(NS * ROWS_PT, tail)])
            pltpu.sync_copy(shden.at[pl.ds(NS * ROWS_PT, tail)],
                            den_h.at[c].at[pl.ds(NS * ROWS_PT, tail)])

    return gat_kernel(row, col, ea, lin, sij, ce16)


BN = 1000
GRID = N // BN

def _blk(cols):
    return pl.BlockSpec((BN, cols), lambda i: (i, 0))

def _full(r, cols):
    return pl.BlockSpec((r, cols), lambda i: (0, 0))


def _dot(a, b):
    return jnp.dot(a, b, preferred_element_type=f32)


def _lin_sij(xn, W, b, gl, ai, aj, Sa, Sb, lin_r, sij_r):
    t = _dot(xn, W[...]) + b[...]
    ln = _dot(t, gl[...])
    lin_r[...] = ln
    sij_r[...] = _dot(ln * ai[...], Sa[...]) + _dot(ln * aj[...], Sb[...])


def _tc1_body(x_r, W_r, b_r, gl_r, ai_r, aj_r, Sa_r, Sb_r, lin_r, sij_r):
    _lin_sij(x_r[...], W_r, b_r, gl_r, ai_r, aj_r, Sa_r, Sb_r, lin_r, sij_r)


def _tc1(x, W, b, gl, ai, aj, Sa, Sb):
    return pl.pallas_call(
        _tc1_body,
        grid=(GRID,),
        in_specs=[_blk(D), _full(D, D), _full(1, D), _full(D, D),
                  _full(1, D), _full(1, D), _full(D, 16), _full(D, 16)],
        out_specs=[_blk(D), _blk(16)],
        out_shape=[jax.ShapeDtypeStruct((N, D), f32),
                   jax.ShapeDtypeStruct((N, 16), f32)],
    )(x, W, b, gl, ai, aj, Sa, Sb)


def _tcmid_body(n0, n1, d0, d1, xp_r, bng_r, bnb_r, Ws_r, bs_r, EXP_r,
                W_r, b_r, gl_r, ai_r, aj_r, Sa_r, Sb_r,
                xn_r, lin_r, sij_r):
    num = n0[...] + n1[...]
    den = _dot(d0[...] + d1[...], EXP_r[...])
    gat = num / (den + 1e-16)
    y = gat * (bng_r[...] * (1.0 / jnp.sqrt(1.0 + 1e-5))) + bnb_r[...]
    y = jnp.maximum(y, 0.0)
    xn = y + _dot(xp_r[...], Ws_r[...]) + bs_r[...]
    xn_r[...] = xn
    _lin_sij(xn, W_r, b_r, gl_r, ai_r, aj_r, Sa_r, Sb_r, lin_r, sij_r)


def _tcmid(n0, n1, d0, d1, xp, bng, bnb, Ws, bs, EXPd, W, b, gl, ai, aj, Sa, Sb):
    return pl.pallas_call(
        _tcmid_body,
        grid=(GRID,),
        in_specs=[_blk(D), _blk(D), _blk(16), _blk(16), _blk(D),
                  _full(1, D), _full(1, D), _full(D, D), _full(1, D),
                  _full(16, D),
                  _full(D, D), _full(1, D), _full(D, D),
                  _full(1, D), _full(1, D), _full(D, 16), _full(D, 16)],
        out_specs=[_blk(D), _blk(D), _blk(16)],
        out_shape=[jax.ShapeDtypeStruct((N, D), f32),
                   jax.ShapeDtypeStruct((N, D), f32),
                   jax.ShapeDtypeStruct((N, 16), f32)],
    )(n0, n1, d0, d1, xp, bng, bnb, Ws, bs, EXPd, W, b, gl, ai, aj, Sa, Sb)


def _tcpost_body(n0, n1, d0, d1, xp_r, lng_r, lnb_r, Ws_r, bs_r, EXP_r, out_r):
    num = n0[...] + n1[...]
    den = _dot(d0[...] + d1[...], EXP_r[...])
    gat = num / (den + 1e-16)
    m = jnp.mean(gat, axis=-1, keepdims=True)
    v = jnp.mean((gat - m) ** 2, axis=-1, keepdims=True)
    y = (gat - m) / jnp.sqrt(v + 1e-5) * lng_r[...] + lnb_r[...]
    y = jnp.maximum(y, 0.0)
    out_r[...] = y + _dot(xp_r[...], Ws_r[...]) + bs_r[...]


def _tcpost(n0, n1, d0, d1, xp, lng, lnb, Ws, bs, EXPd):
    return pl.pallas_call(
        _tcpost_body,
        grid=(GRID,),
        in_specs=[_blk(D), _blk(D), _blk(16), _blk(16), _blk(D),
                  _full(1, D), _full(1, D), _full(D, D), _full(1, D),
                  _full(16, D)],
        out_specs=_blk(D),
        out_shape=jax.ShapeDtypeStruct((N, D), f32),
    )(n0, n1, d0, d1, xp, lng, lnb, Ws, bs, EXPd)


def kernel(x, edge_index, edge_attr, g1_lin, g1_edge, g1_att, g2_lin, g2_edge,
           g2_att, g3_lin, g3_edge, g3_att, W1, b1, W2, b2, W3, b3, Ws1, bs1,
           Ws2, bs2, Ws3, bs3, bn1_g, bn1_b, bn2_g, bn2_b, ln_g, ln_b):
    row = edge_index[0]
    col = edge_index[1]

    r1 = lambda v: v.reshape(1, D)

    ai1 = g1_att[0, :, :OC].reshape(1, D)
    aj1 = g1_att[0, :, OC:2 * OC].reshape(1, D)
    ce1 = (g1_edge.reshape(H, OC) * g1_att[0, :, 2 * OC:]).sum(-1)
    ce16_1 = jnp.tile(ce1, 2)
    ai2 = g2_att[0, :, :OC].reshape(1, D)
    aj2 = g2_att[0, :, OC:2 * OC].reshape(1, D)
    ce2 = (g2_edge.reshape(H, OC) * g2_att[0, :, 2 * OC:]).sum(-1)
    ce16_2 = jnp.tile(ce2, 2)
    hh = jnp.arange(D, dtype=jnp.int32) // OC
    Sa = jax.nn.one_hot(hh, 16, dtype=f32)
    Sb = jax.nn.one_hot(hh + 8, 16, dtype=f32)
    EXPd = Sa.T

    ai3 = g3_att[0, 0, :D].reshape(1, D)
    aj3 = g3_att[0, 0, D:2 * D].reshape(1, D)
    ce3 = (g3_edge[0] * g3_att[0, 0, 2 * D:]).sum()
    ce16_3 = jnp.full((16,), ce3, f32)
    z128 = jnp.zeros((D,), jnp.int32)
    Sa3 = jax.nn.one_hot(z128, 16, dtype=f32)
    Sb3 = jax.nn.one_hot(z128 + 1, 16, dtype=f32)
    EXP3 = Sa3.T

    lin1, sij1 = _tc1(x, W1, r1(b1), g1_lin, ai1, aj1, Sa, Sb)
    num1, den1 = _sc_gat(row, col, edge_attr, lin1, sij1, ce16_1, True)
    x1, lin2, sij2 = _tcmid(num1[0], num1[1], den1[0], den1[1], x,
                            r1(bn1_g), r1(bn1_b), Ws1, r1(bs1), EXPd,
                            W2, r1(b2), g2_lin, ai2, aj2, Sa, Sb)
    num2, den2 = _sc_gat(row, col, edge_attr, lin2, sij2, ce16_2, True)
    x2, lin3, sij3 = _tcmid(num2[0], num2[1], den2[0], den2[1], x1,
                            r1(bn2_g), r1(bn2_b), Ws2, r1(bs2), EXPd,
                            W3, r1(b3), g3_lin, ai3, aj3, Sa3, Sb3)
    num3, den3 = _sc_gat(row, col, edge_attr, lin3, sij3, ce16_3, False)
    out = _tcpost(num3[0], num3[1], den3[0], den3[1], x2,
                  r1(ln_g), r1(ln_b), Ws3, r1(bs3), EXP3)
    return out


# speedup vs baseline: 58.6569x; 1.3941x over previous
import functools
import jax
import jax.numpy as jnp
from jax import lax
from jax.experimental import pallas as pl
from jax.experimental.pallas import tpu as pltpu
from jax.experimental.pallas import tpu_sc as plsc

N = 10000
E = 320000
D = 128
H = 8
OC = 16

NC = 2
NS = 16
DH = D // NC
EPT = E // NS
K = 80
NCHUNK = EPT // K
MW = DH + 16
ROWS_PT = 624

f32 = jnp.float32


def _bcast_lane(v, k):
    idx = jnp.full((16, 1), k, jnp.int32)
    return lax.gather(
        v, idx,
        dimension_numbers=lax.GatherDimensionNumbers(
            offset_dims=(), collapsed_slice_dims=(0,), start_index_map=(0,)),
        slice_sizes=(1,),
        mode=lax.GatherScatterMode.PROMISE_IN_BOUNDS)


def _sc_gat(row3, col3, ea3, lin_hs, sij_hs, ce16, multi_head):
    mesh = plsc.VectorSubcoreMesh(
        core_axis_name="c", subcore_axis_name="s", num_cores=NC, num_subcores=NS)

    buf8 = pltpu.VMEM((K, 8), f32)
    bufH = pltpu.VMEM((K, DH), f32)
    bufM = pltpu.VMEM((K, MW), f32)

    @functools.partial(
        pl.kernel,
        out_type=jax.ShapeDtypeStruct((NC, N, MW), f32),
        mesh=mesh,
        compiler_params=pltpu.CompilerParams(needs_layout_passes=False,
                                             use_tc_tiling_on_sc=False),
        scratch_types=[
            pltpu.VMEM((NCHUNK, K), jnp.int32),
            pltpu.VMEM((NCHUNK, K), jnp.int32),
            pltpu.VMEM((K,), f32), pltpu.VMEM((K,), f32),
            buf8, buf8,
            buf8, buf8,
            bufH, bufH,
            bufM, bufM,
            pltpu.VMEM((16,), f32),
            pltpu.VMEM_SHARED((N, MW), f32),
            pltpu.SemaphoreType.DMA,
            pltpu.SemaphoreType.DMA,
            pltpu.SemaphoreType.DMA,
            pltpu.SemaphoreType.DMA,
        ],
    )
    def gat_kernel(row_h, col_h, ea_h, lin_h, sij_h, ce_h, num_h,
                   idxr_all, idxc_all, eav0, eav1,
                   sr0, sr1, scb0, scb1, linb0, linb1,
                   mrg0, mrg1, cev, shacc,
                   sem_in0, sem_in1, sem_out0, sem_out1):
        c = lax.axis_index("c")
        s = lax.axis_index("s")
        io = lax.iota(jnp.int32, 16)
        zf = jnp.zeros((16,), f32)

        slots = ((sr0, scb0, linb0, mrg0, eav0, sem_in0, sem_out0),
                 (sr1, scb1, linb1, mrg1, eav1, sem_in1, sem_out1))

        pltpu.sync_copy(row_h.at[s], idxr_all)
        pltpu.sync_copy(col_h.at[s], idxc_all)
        pltpu.sync_copy(ce_h, cev)

        @pl.loop(0, K)
        def _zero(r):
            for kk in range(MW // 16):
                mrg0[r, pl.ds(kk * 16, 16)] = zf
                mrg1[r, pl.ds(kk * 16, 16)] = zf

        rb = pl.multiple_of(s * ROWS_PT, 8)
        for t in range(ROWS_PT // K):
            pltpu.sync_copy(mrg0, shacc.at[pl.ds(rb + t * K, K)])
        rem = ROWS_PT % K
        if rem:
            pltpu.sync_copy(mrg0.at[pl.ds(0, rem)],
                            shacc.at[pl.ds(rb + (ROWS_PT // K) * K, rem)])
        tail = N - NS * ROWS_PT

        @pl.when(s == NS - 1)
        def _zero_tail():
            pltpu.sync_copy(mrg0.at[pl.ds(0, tail)],
                            shacc.at[pl.ds(NS * ROWS_PT, tail)])

        plsc.subcore_barrier()

        cb4 = c * 4
        hsel = io & 3
        qsel = io >> 2
        cv = plsc.load_gather(cev, [cb4 + hsel])

        def in_descs(q, slot):
            sr, scb, linb, _, eav, sem_in, _ = slot
            ir = idxr_all.at[q]
            ic = idxc_all.at[q]
            return (pltpu.make_async_copy(sij_h.at[c].at[ir], sr, sem_in),
                    pltpu.make_async_copy(sij_h.at[c].at[ic], scb, sem_in),
                    pltpu.make_async_copy(lin_h.at[c].at[ic], linb, sem_in),
                    pltpu.make_async_copy(ea_h.at[s].at[q], eav, sem_in))

        def start_out(q, slot):
            mrg, sem_out = slot[3], slot[6]
            pltpu.async_copy(mrg, shacc.at[idxr_all.at[q]], sem_out, add=True)

        def wait_out(q, slot):
            mrg, sem_out = slot[3], slot[6]
            pltpu.make_async_copy(mrg, shacc.at[idxr_all.at[q]], sem_out).wait()

        def compute(q, slot):
            sr, scb, linb, mrg, eav, _, _ = slot
            if multi_head:
                @pl.loop(0, K // 4, unroll=10)
                def _quad(i):
                    e4 = i * 4
                    rsel = e4 + qsel
                    si_v = plsc.load_gather(sr, [rsel, hsel])
                    sj_v = plsc.load_gather(scb, [rsel, hsel + 4])
                    ea_v = plsc.load_gather(eav, [rsel])
                    al = si_v + sj_v + ea_v * cv
                    al = jnp.maximum(al, 0.2 * al)
                    ex = jnp.exp(al)
                    plsc.store_scatter(mrg, [rsel, DH + cb4 + hsel], ex)
                    for l in range(4):
                        for kk in range(4):
                            sl = pl.ds(kk * 16, 16)
                            mrg[e4 + l, sl] = (linb[e4 + l, sl]
                                               * _bcast_lane(ex, l * 4 + kk))
            else:
                io0 = io * 0

                @pl.loop(0, K // 16, unroll=5)
                def _grp(j):
                    e0 = j * 16
                    rows = e0 + io
                    si_v = plsc.load_gather(sr, [rows, io0])
                    sj_v = plsc.load_gather(scb, [rows, io0 + 4])
                    ea_v = eav[pl.ds(e0, 16)]
                    al = si_v + sj_v + ea_v * cv
                    al = jnp.maximum(al, 0.2 * al)
                    ex = jnp.exp(al)
                    plsc.store_scatter(mrg, [rows, io0 + DH], ex)
                    for l in range(16):
                        m = _bcast_lane(ex, l)
                        for kk in range(DH // 16):
                            sl = pl.ds(kk * 16, 16)
                            mrg[e0 + l, sl] = linb[e0 + l, sl] * m

        for d in in_descs(0, slots[0]):
            d.start()
        for d in in_descs(1, slots[1]):
            d.start()

        @pl.loop(0, NCHUNK // 2)
        def _body(t):
            for j in range(2):
                q = t * 2 + j
                slot = slots[j]
                for d in in_descs(q, slot):
                    d.wait()

                @pl.when(t > 0)
                def _drain():
                    wait_out(q, slot)

                compute(q, slot)
                start_out(q, slot)

                @pl.when(q + 2 < NCHUNK)
                def _refill():
                    for d in in_descs(q + 2, slot):
                        d.start()

        wait_out(0, slots[0])
        wait_out(1, slots[1])

        plsc.subcore_barrier()
        pltpu.sync_copy(shacc.at[pl.ds(rb, ROWS_PT)],
                        num_h.at[c].at[pl.ds(rb, ROWS_PT)])

        @pl.when(s == NS - 1)
        def _copy_tail():
            pltpu.sync_copy(shacc.at[pl.ds(NS * ROWS_PT, tail)],
                            num_h.at[c].at[pl.ds(NS * ROWS_PT, tail)])

    return gat_kernel(row3, col3, ea3, lin_hs, sij_hs, ce16)


BN = 1000
GRID = N // BN

def _blk(cols):
    return pl.BlockSpec((BN, cols), lambda i: (i, 0))

def _blk3(cols):
    return pl.BlockSpec((NC, BN, cols), lambda i: (0, i, 0))

def _full(r, cols):
    return pl.BlockSpec((r, cols), lambda i: (0, 0))


def _dot(a, b):
    return jnp.dot(a, b, preferred_element_type=f32)


def _lin_sij(xn, W, b, gl, ai, aj, SaA, SbA, SaB, SbB, lin_r, sij_r):
    t = _dot(xn, W[...]) + b[...]
    ln = _dot(t, gl[...])
    lin_r[0] = ln[:, :DH]
    lin_r[1] = ln[:, DH:]
    p = ln * ai[...]
    q = ln * aj[...]
    sij_r[0] = _dot(p, SaA[...]) + _dot(q, SbA[...])
    sij_r[1] = _dot(p, SaB[...]) + _dot(q, SbB[...])


def _tc1_body(x_r, W_r, b_r, gl_r, ai_r, aj_r, SaA_r, SbA_r, SaB_r, SbB_r,
              lin_r, sij_r):
    _lin_sij(x_r[...], W_r, b_r, gl_r, ai_r, aj_r, SaA_r, SbA_r, SaB_r, SbB_r,
             lin_r, sij_r)


def _tc1(x, W, b, gl, ai, aj, SaA, SbA, SaB, SbB):
    return pl.pallas_call(
        _tc1_body,
        grid=(GRID,),
        in_specs=[_blk(D), _full(D, D), _full(1, D), _full(D, D),
                  _full(1, D), _full(1, D),
                  _full(D, 8), _full(D, 8), _full(D, 8), _full(D, 8)],
        out_specs=[_blk3(DH), _blk3(8)],
        out_shape=[jax.ShapeDtypeStruct((NC, N, DH), f32),
                   jax.ShapeDtypeStruct((NC, N, 8), f32)],
    )(x, W, b, gl, ai, aj, SaA, SbA, SaB, SbB)


def _gat_combine(n_r, EXP_r):
    n0 = n_r[0]
    n1 = n_r[1]
    num = jnp.concatenate([n0[:, :DH], n1[:, :DH]], axis=-1)
    den = _dot(n0[:, DH:] + n1[:, DH:], EXP_r[...])
    return num / (den + 1e-16)


def _tcmid_body(n_r, xp_r, bng_r, bnb_r, Ws_r, bs_r, EXP_r,
                W_r, b_r, gl_r, ai_r, aj_r, SaA_r, SbA_r, SaB_r, SbB_r,
                xn_r, lin_r, sij_r):
    gat = _gat_combine(n_r, EXP_r)
    y = gat * (bng_r[...] * (1.0 / jnp.sqrt(1.0 + 1e-5))) + bnb_r[...]
    y = jnp.maximum(y, 0.0)
    xn = y + _dot(xp_r[...], Ws_r[...]) + bs_r[...]
    xn_r[...] = xn
    _lin_sij(xn, W_r, b_r, gl_r, ai_r, aj_r, SaA_r, SbA_r, SaB_r, SbB_r,
             lin_r, sij_r)


def _tcmid(num, xp, bng, bnb, Ws, bs, EXPd,
           W, b, gl, ai, aj, SaA, SbA, SaB, SbB):
    return pl.pallas_call(
        _tcmid_body,
        grid=(GRID,),
        in_specs=[_blk3(MW), _blk(D),
                  _full(1, D), _full(1, D), _full(D, D), _full(1, D),
                  _full(16, D),
                  _full(D, D), _full(1, D), _full(D, D),
                  _full(1, D), _full(1, D),
                  _full(D, 8), _full(D, 8), _full(D, 8), _full(D, 8)],
        out_specs=[_blk(D), _blk3(DH), _blk3(8)],
        out_shape=[jax.ShapeDtypeStruct((N, D), f32),
                   jax.ShapeDtypeStruct((NC, N, DH), f32),
                   jax.ShapeDtypeStruct((NC, N, 8), f32)],
    )(num, xp, bng, bnb, Ws, bs, EXPd, W, b, gl, ai, aj,
      SaA, SbA, SaB, SbB)


def _tcpost_body(n_r, xp_r, lng_r, lnb_r, Ws_r, bs_r, EXP_r, out_r):
    gat = _gat_combine(n_r, EXP_r)
    m = jnp.mean(gat, axis=-1, keepdims=True)
    v = jnp.mean((gat - m) ** 2, axis=-1, keepdims=True)
    y = (gat - m) / jnp.sqrt(v + 1e-5) * lng_r[...] + lnb_r[...]
    y = jnp.maximum(y, 0.0)
    out_r[...] = y + _dot(xp_r[...], Ws_r[...]) + bs_r[...]


def _tcpost(num, xp, lng, lnb, Ws, bs, EXPd):
    return pl.pallas_call(
        _tcpost_body,
        grid=(GRID,),
        in_specs=[_blk3(MW), _blk(D),
                  _full(1, D), _full(1, D), _full(D, D), _full(1, D),
                  _full(16, D)],
        out_specs=_blk(D),
        out_shape=jax.ShapeDtypeStruct((N, D), f32),
    )(num, xp, lng, lnb, Ws, bs, EXPd)


def kernel(x, edge_index, edge_attr, g1_lin, g1_edge, g1_att, g2_lin, g2_edge,
           g2_att, g3_lin, g3_edge, g3_att, W1, b1, W2, b2, W3, b3, Ws1, bs1,
           Ws2, bs2, Ws3, bs3, bn1_g, bn1_b, bn2_g, bn2_b, ln_g, ln_b):
    row3 = edge_index[0].reshape(NS, NCHUNK, K)
    col3 = edge_index[1].reshape(NS, NCHUNK, K)
    ea3 = edge_attr.reshape(NS, NCHUNK, K)

    r1 = lambda v: v.reshape(1, D)
    hh = jnp.arange(D, dtype=jnp.int32) // OC
    z4 = jnp.zeros((D, 4), f32)
    oh = lambda idx: jax.nn.one_hot(idx, 4, dtype=f32)
    SaA = jnp.concatenate([oh(hh), z4], axis=1)
    SbA = jnp.concatenate([z4, oh(hh)], axis=1)
    SaB = jnp.concatenate([oh(hh - 4), z4], axis=1)
    SbB = jnp.concatenate([z4, oh(hh - 4)], axis=1)
    EXPd = jax.nn.one_hot(hh, 16, dtype=f32).T

    ai1 = g1_att[0, :, :OC].reshape(1, D)
    aj1 = g1_att[0, :, OC:2 * OC].reshape(1, D)
    ce1 = (g1_edge.reshape(H, OC) * g1_att[0, :, 2 * OC:]).sum(-1)
    ce16_1 = jnp.tile(ce1, 2)
    ai2 = g2_att[0, :, :OC].reshape(1, D)
    aj2 = g2_att[0, :, OC:2 * OC].reshape(1, D)
    ce2 = (g2_edge.reshape(H, OC) * g2_att[0, :, 2 * OC:]).sum(-1)
    ce16_2 = jnp.tile(ce2, 2)

    ai3 = g3_att[0, 0, :D].reshape(1, D)
    aj3 = g3_att[0, 0, D:2 * D].reshape(1, D)
    ce3 = (g3_edge[0] * g3_att[0, 0, 2 * D:]).sum()
    ce16_3 = jnp.full((16,), ce3, f32)
    z128 = jnp.zeros((D,), jnp.int32)
    Sa3 = jax.nn.one_hot(z128, 8, dtype=f32)
    Sb3 = jax.nn.one_hot(z128 + 4, 8, dtype=f32)
    EXP3 = 0.5 * jax.nn.one_hot(z128, 16, dtype=f32).T

    lin1, sij1 = _tc1(x, W1, r1(b1), g1_lin, ai1, aj1, SaA, SbA, SaB, SbB)
    num1 = _sc_gat(row3, col3, ea3, lin1, sij1, ce16_1, True)
    x1, lin2, sij2 = _tcmid(num1, x,
                            r1(bn1_g), r1(bn1_b), Ws1, r1(bs1), EXPd,
                            W2, r1(b2), g2_lin, ai2, aj2, SaA, SbA, SaB, SbB)
    num2 = _sc_gat(row3, col3, ea3, lin2, sij2, ce16_2, True)
    x2, lin3, sij3 = _tcmid(num2, x1,
                            r1(bn2_g), r1(bn2_b), Ws2, r1(bs2), EXPd,
                            W3, r1(b3), g3_lin, ai3, aj3, Sa3, Sb3, Sa3, Sb3)
    num3 = _sc_gat(row3, col3, ea3, lin3, sij3, ce16_3, False)
    out = _tcpost(num3, x2, r1(ln_g), r1(ln_b), Ws3, r1(bs3), EXP3)
    return out


# speedup vs baseline: 116.7413x; 1.9902x over previous
import functools
import jax
import jax.numpy as jnp
from jax import lax
from jax.experimental import pallas as pl
from jax.experimental.pallas import tpu as pltpu
from jax.experimental.pallas import tpu_sc as plsc

N = 10000
E = 320000
D = 128
H = 8
OC = 16

NC = 2
NS = 16
DH = D // NC
EPT = E // NS
K = 80
NCHUNK = EPT // K
MW = DH + 16
ROWS_PT = 624

f32 = jnp.float32


def _bcast_lane(v, k):
    idx = jnp.full((16, 1), k, jnp.int32)
    return lax.gather(
        v, idx,
        dimension_numbers=lax.GatherDimensionNumbers(
            offset_dims=(), collapsed_slice_dims=(0,), start_index_map=(0,)),
        slice_sizes=(1,),
        mode=lax.GatherScatterMode.PROMISE_IN_BOUNDS)


def _sc_gat(row3, col3, ea3, lin_hs, sij_hs, ce16, multi_head):
    mesh = plsc.VectorSubcoreMesh(
        core_axis_name="c", subcore_axis_name="s", num_cores=NC, num_subcores=NS)

    buf8 = pltpu.VMEM((K, 8), f32)
    bufH = pltpu.VMEM((K, DH), f32)
    bufM = pltpu.VMEM((K, MW), f32)

    @functools.partial(
        pl.kernel,
        out_type=jax.ShapeDtypeStruct((NC, N, MW), f32),
        mesh=mesh,
        compiler_params=pltpu.CompilerParams(needs_layout_passes=False,
                                             use_tc_tiling_on_sc=False),
        scratch_types=[
            pltpu.VMEM((NCHUNK, K), jnp.int32),
            pltpu.VMEM((NCHUNK, K), jnp.int32),
            pltpu.VMEM((K,), f32), pltpu.VMEM((K,), f32),
            buf8, buf8,
            buf8, buf8,
            bufH, bufH,
            bufM, bufM,
            pltpu.VMEM((16,), f32),
            pltpu.VMEM_SHARED((N, MW), f32),
            pltpu.SemaphoreType.DMA,
            pltpu.SemaphoreType.DMA,
            pltpu.SemaphoreType.DMA,
            pltpu.SemaphoreType.DMA,
        ],
    )
    def gat_kernel(row_h, col_h, ea_h, lin_h, sij_h, ce_h, num_h,
                   idxr_all, idxc_all, eav0, eav1,
                   sr0, sr1, scb0, scb1, linb0, linb1,
                   mrg0, mrg1, cev, shacc,
                   sem_in0, sem_in1, sem_out0, sem_out1):
        c = lax.axis_index("c")
        s = lax.axis_index("s")
        io = lax.iota(jnp.int32, 16)
        zf = jnp.zeros((16,), f32)

        slots = ((sr0, scb0, linb0, mrg0, eav0, sem_in0, sem_out0),
                 (sr1, scb1, linb1, mrg1, eav1, sem_in1, sem_out1))

        pltpu.sync_copy(row_h.at[s], idxr_all)
        pltpu.sync_copy(col_h.at[s], idxc_all)
        pltpu.sync_copy(ce_h, cev)

        @pl.loop(0, K)
        def _zero(r):
            for kk in range(MW // 16):
                mrg0[r, pl.ds(kk * 16, 16)] = zf
                mrg1[r, pl.ds(kk * 16, 16)] = zf

        rb = pl.multiple_of(s * ROWS_PT, 8)
        for t in range(ROWS_PT // K):
            pltpu.sync_copy(mrg0, shacc.at[pl.ds(rb + t * K, K)])
        rem = ROWS_PT % K
        if rem:
            pltpu.sync_copy(mrg0.at[pl.ds(0, rem)],
                            shacc.at[pl.ds(rb + (ROWS_PT // K) * K, rem)])
        tail = N - NS * ROWS_PT

        @pl.when(s == NS - 1)
        def _zero_tail():
            pltpu.sync_copy(mrg0.at[pl.ds(0, tail)],
                            shacc.at[pl.ds(NS * ROWS_PT, tail)])

        plsc.subcore_barrier()

        cb4 = c * 4
        hsel = io & 3
        qsel = io >> 2
        cv = plsc.load_gather(cev, [cb4 + hsel])

        def in_descs(q, slot):
            sr, scb, linb, _, eav, sem_in, _ = slot
            ir = idxr_all.at[q]
            ic = idxc_all.at[q]
            return (pltpu.make_async_copy(sij_h.at[c].at[ir], sr, sem_in),
                    pltpu.make_async_copy(sij_h.at[c].at[ic], scb, sem_in),
                    pltpu.make_async_copy(lin_h.at[c].at[ic], linb, sem_in),
                    pltpu.make_async_copy(ea_h.at[s].at[q], eav, sem_in))

        def start_out(q, slot):
            mrg, sem_out = slot[3], slot[6]
            pltpu.async_copy(mrg, shacc.at[idxr_all.at[q]], sem_out, add=True)

        def wait_out(q, slot):
            mrg, sem_out = slot[3], slot[6]
            pltpu.make_async_copy(mrg, shacc.at[idxr_all.at[q]], sem_out).wait()

        def compute(q, slot):
            sr, scb, linb, mrg, eav, _, _ = slot
            if multi_head:
                @plsc.parallel_loop(0, K // 4, unroll=4)
                def _quad(i):
                    e4 = i * 4
                    rsel = e4 + qsel
                    si_v = plsc.load_gather(sr, [rsel, hsel])
                    sj_v = plsc.load_gather(scb, [rsel, hsel + 4])
                    ea_v = plsc.load_gather(eav, [rsel])
                    al = si_v + sj_v + ea_v * cv
                    al = jnp.maximum(al, 0.2 * al)
                    ex = jnp.exp(al)
                    plsc.store_scatter(mrg, [rsel, DH + cb4 + hsel], ex)
                    for l in range(4):
                        for kk in range(4):
                            sl = pl.ds(kk * 16, 16)
                            mrg[e4 + l, sl] = (linb[e4 + l, sl]
                                               * _bcast_lane(ex, l * 4 + kk))
            else:
                io0 = io * 0

                @plsc.parallel_loop(0, K // 16, unroll=5)
                def _grp(j):
                    e0 = j * 16
                    rows = e0 + io
                    si_v = plsc.load_gather(sr, [rows, io0])
                    sj_v = plsc.load_gather(scb, [rows, io0 + 4])
                    ea_v = eav[pl.ds(e0, 16)]
                    al = si_v + sj_v + ea_v * cv
                    al = jnp.maximum(al, 0.2 * al)
                    ex = jnp.exp(al)
                    plsc.store_scatter(mrg, [rows, io0 + DH], ex)
                    for l in range(16):
                        m = _bcast_lane(ex, l)
                        for kk in range(DH // 16):
                            sl = pl.ds(kk * 16, 16)
                            mrg[e0 + l, sl] = linb[e0 + l, sl] * m

        for d in in_descs(0, slots[0]):
            d.start()
        for d in in_descs(1, slots[1]):
            d.start()

        @pl.loop(0, NCHUNK // 2)
        def _body(t):
            for j in range(2):
                q = t * 2 + j
                slot = slots[j]
                for d in in_descs(q, slot):
                    d.wait()

                @pl.when(t > 0)
                def _drain():
                    wait_out(q, slot)

                compute(q, slot)
                start_out(q, slot)

                @pl.when(q + 2 < NCHUNK)
                def _refill():
                    for d in in_descs(q + 2, slot):
                        d.start()

        wait_out(0, slots[0])
        wait_out(1, slots[1])

        plsc.subcore_barrier()
        pltpu.sync_copy(shacc.at[pl.ds(rb, ROWS_PT)],
                        num_h.at[c].at[pl.ds(rb, ROWS_PT)])

        @pl.when(s == NS - 1)
        def _copy_tail():
            pltpu.sync_copy(shacc.at[pl.ds(NS * ROWS_PT, tail)],
                            num_h.at[c].at[pl.ds(NS * ROWS_PT, tail)])

    return gat_kernel(row3, col3, ea3, lin_hs, sij_hs, ce16)


BN = 1000
GRID = N // BN

def _blk(cols):
    return pl.BlockSpec((BN, cols), lambda i: (i, 0))

def _blk3(cols):
    return pl.BlockSpec((NC, BN, cols), lambda i: (0, i, 0))

def _full(r, cols):
    return pl.BlockSpec((r, cols), lambda i: (0, 0))


def _dot(a, b):
    return jnp.dot(a, b, preferred_element_type=f32)


def _lin_sij(xn, W, b, gl, ai, aj, SaA, SbA, SaB, SbB, lin_r, sij_r):
    t = _dot(xn, W[...]) + b[...]
    ln = _dot(t, gl[...])
    lin_r[0] = ln[:, :DH]
    lin_r[1] = ln[:, DH:]
    p = ln * ai[...]
    q = ln * aj[...]
    sij_r[0] = _dot(p, SaA[...]) + _dot(q, SbA[...])
    sij_r[1] = _dot(p, SaB[...]) + _dot(q, SbB[...])


def _tc1_body(x_r, W_r, b_r, gl_r, ai_r, aj_r, SaA_r, SbA_r, SaB_r, SbB_r,
              lin_r, sij_r):
    _lin_sij(x_r[...], W_r, b_r, gl_r, ai_r, aj_r, SaA_r, SbA_r, SaB_r, SbB_r,
             lin_r, sij_r)


def _tc1(x, W, b, gl, ai, aj, SaA, SbA, SaB, SbB):
    return pl.pallas_call(
        _tc1_body,
        grid=(GRID,),
        in_specs=[_blk(D), _full(D, D), _full(1, D), _full(D, D),
                  _full(1, D), _full(1, D),
                  _full(D, 8), _full(D, 8), _full(D, 8), _full(D, 8)],
        out_specs=[_blk3(DH), _blk3(8)],
        out_shape=[jax.ShapeDtypeStruct((NC, N, DH), f32),
                   jax.ShapeDtypeStruct((NC, N, 8), f32)],
    )(x, W, b, gl, ai, aj, SaA, SbA, SaB, SbB)


def _gat_combine(n_r, EXP_r):
    n0 = n_r[0]
    n1 = n_r[1]
    num = jnp.concatenate([n0[:, :DH], n1[:, :DH]], axis=-1)
    den = _dot(n0[:, DH:] + n1[:, DH:], EXP_r[...])
    return num / (den + 1e-16)


def _tcmid_body(n_r, xp_r, bng_r, bnb_r, Ws_r, bs_r, EXP_r,
                W_r, b_r, gl_r, ai_r, aj_r, SaA_r, SbA_r, SaB_r, SbB_r,
                xn_r, lin_r, sij_r):
    gat = _gat_combine(n_r, EXP_r)
    y = gat * (bng_r[...] * (1.0 / jnp.sqrt(1.0 + 1e-5))) + bnb_r[...]
    y = jnp.maximum(y, 0.0)
    xn = y + _dot(xp_r[...], Ws_r[...]) + bs_r[...]
    xn_r[...] = xn
    _lin_sij(xn, W_r, b_r, gl_r, ai_r, aj_r, SaA_r, SbA_r, SaB_r, SbB_r,
             lin_r, sij_r)


def _tcmid(num, xp, bng, bnb, Ws, bs, EXPd,
           W, b, gl, ai, aj, SaA, SbA, SaB, SbB):
    return pl.pallas_call(
        _tcmid_body,
        grid=(GRID,),
        in_specs=[_blk3(MW), _blk(D),
                  _full(1, D), _full(1, D), _full(D, D), _full(1, D),
                  _full(16, D),
                  _full(D, D), _full(1, D), _full(D, D),
                  _full(1, D), _full(1, D),
                  _full(D, 8), _full(D, 8), _full(D, 8), _full(D, 8)],
        out_specs=[_blk(D), _blk3(DH), _blk3(8)],
        out_shape=[jax.ShapeDtypeStruct((N, D), f32),
                   jax.ShapeDtypeStruct((NC, N, DH), f32),
                   jax.ShapeDtypeStruct((NC, N, 8), f32)],
    )(num, xp, bng, bnb, Ws, bs, EXPd, W, b, gl, ai, aj,
      SaA, SbA, SaB, SbB)


def _tcpost_body(n_r, xp_r, lng_r, lnb_r, Ws_r, bs_r, EXP_r, out_r):
    gat = _gat_combine(n_r, EXP_r)
    m = jnp.mean(gat, axis=-1, keepdims=True)
    v = jnp.mean((gat - m) ** 2, axis=-1, keepdims=True)
    y = (gat - m) / jnp.sqrt(v + 1e-5) * lng_r[...] + lnb_r[...]
    y = jnp.maximum(y, 0.0)
    out_r[...] = y + _dot(xp_r[...], Ws_r[...]) + bs_r[...]


def _tcpost(num, xp, lng, lnb, Ws, bs, EXPd):
    return pl.pallas_call(
        _tcpost_body,
        grid=(GRID,),
        in_specs=[_blk3(MW), _blk(D),
                  _full(1, D), _full(1, D), _full(D, D), _full(1, D),
                  _full(16, D)],
        out_specs=_blk(D),
        out_shape=jax.ShapeDtypeStruct((N, D), f32),
    )(num, xp, lng, lnb, Ws, bs, EXPd)


def kernel(x, edge_index, edge_attr, g1_lin, g1_edge, g1_att, g2_lin, g2_edge,
           g2_att, g3_lin, g3_edge, g3_att, W1, b1, W2, b2, W3, b3, Ws1, bs1,
           Ws2, bs2, Ws3, bs3, bn1_g, bn1_b, bn2_g, bn2_b, ln_g, ln_b):
    row3 = edge_index[0].reshape(NS, NCHUNK, K)
    col3 = edge_index[1].reshape(NS, NCHUNK, K)
    ea3 = edge_attr.reshape(NS, NCHUNK, K)

    r1 = lambda v: v.reshape(1, D)
    hh = jnp.arange(D, dtype=jnp.int32) // OC
    z4 = jnp.zeros((D, 4), f32)
    oh = lambda idx: jax.nn.one_hot(idx, 4, dtype=f32)
    SaA = jnp.concatenate([oh(hh), z4], axis=1)
    SbA = jnp.concatenate([z4, oh(hh)], axis=1)
    SaB = jnp.concatenate([oh(hh - 4), z4], axis=1)
    SbB = jnp.concatenate([z4, oh(hh - 4)], axis=1)
    EXPd = jax.nn.one_hot(hh, 16, dtype=f32).T

    ai1 = g1_att[0, :, :OC].reshape(1, D)
    aj1 = g1_att[0, :, OC:2 * OC].reshape(1, D)
    ce1 = (g1_edge.reshape(H, OC) * g1_att[0, :, 2 * OC:]).sum(-1)
    ce16_1 = jnp.tile(ce1, 2)
    ai2 = g2_att[0, :, :OC].reshape(1, D)
    aj2 = g2_att[0, :, OC:2 * OC].reshape(1, D)
    ce2 = (g2_edge.reshape(H, OC) * g2_att[0, :, 2 * OC:]).sum(-1)
    ce16_2 = jnp.tile(ce2, 2)

    ai3 = g3_att[0, 0, :D].reshape(1, D)
    aj3 = g3_att[0, 0, D:2 * D].reshape(1, D)
    ce3 = (g3_edge[0] * g3_att[0, 0, 2 * D:]).sum()
    ce16_3 = jnp.full((16,), ce3, f32)
    z128 = jnp.zeros((D,), jnp.int32)
    Sa3 = jax.nn.one_hot(z128, 8, dtype=f32)
    Sb3 = jax.nn.one_hot(z128 + 4, 8, dtype=f32)
    EXP3 = 0.5 * jax.nn.one_hot(z128, 16, dtype=f32).T

    lin1, sij1 = _tc1(x, W1, r1(b1), g1_lin, ai1, aj1, SaA, SbA, SaB, SbB)
    num1 = _sc_gat(row3, col3, ea3, lin1, sij1, ce16_1, True)
    x1, lin2, sij2 = _tcmid(num1, x,
                            r1(bn1_g), r1(bn1_b), Ws1, r1(bs1), EXPd,
                            W2, r1(b2), g2_lin, ai2, aj2, SaA, SbA, SaB, SbB)
    num2 = _sc_gat(row3, col3, ea3, lin2, sij2, ce16_2, True)
    x2, lin3, sij3 = _tcmid(num2, x1,
                            r1(bn2_g), r1(bn2_b), Ws2, r1(bs2), EXPd,
                            W3, r1(b3), g3_lin, ai3, aj3, Sa3, Sb3, Sa3, Sb3)
    num3 = _sc_gat(row3, col3, ea3, lin3, sij3, ce16_3, False)
    out = _tcpost(num3, x2, r1(ln_g), r1(ln_b), Ws3, r1(bs3), EXP3)
    return out


# speedup vs baseline: 127.2633x; 1.0901x over previous
import functools
import jax
import jax.numpy as jnp
from jax import lax
from jax.experimental import pallas as pl
from jax.experimental.pallas import tpu as pltpu
from jax.experimental.pallas import tpu_sc as plsc

N = 10000
E = 320000
D = 128
H = 8
OC = 16

NC = 2
NS = 16
DH = D // NC
EPT = E // NS
K = 80
NCHUNK = EPT // K
MW = DH + 16
ROWS_PT = 624

f32 = jnp.float32


def _bcast_lane(v, k):
    idx = jnp.full((16, 1), k, jnp.int32)
    return lax.gather(
        v, idx,
        dimension_numbers=lax.GatherDimensionNumbers(
            offset_dims=(), collapsed_slice_dims=(0,), start_index_map=(0,)),
        slice_sizes=(1,),
        mode=lax.GatherScatterMode.PROMISE_IN_BOUNDS)


def _sc_gat(row3, col3, ea3, lin_hs, sij_hs, ce16, multi_head):
    mesh = plsc.VectorSubcoreMesh(
        core_axis_name="c", subcore_axis_name="s", num_cores=NC, num_subcores=NS)

    buf8 = pltpu.VMEM((K, 8), f32)
    bufH = pltpu.VMEM((K, DH), jnp.bfloat16)
    bufM = pltpu.VMEM((K, MW), f32)

    @functools.partial(
        pl.kernel,
        out_type=jax.ShapeDtypeStruct((NC, N, MW), f32),
        mesh=mesh,
        compiler_params=pltpu.CompilerParams(needs_layout_passes=False,
                                             use_tc_tiling_on_sc=False),
        scratch_types=[
            pltpu.VMEM((NCHUNK, K), jnp.int32),
            pltpu.VMEM((NCHUNK, K), jnp.int32),
            pltpu.VMEM((K,), f32), pltpu.VMEM((K,), f32),
            buf8, buf8,
            buf8, buf8,
            bufH, bufH,
            bufM, bufM,
            pltpu.VMEM((16,), f32),
            pltpu.VMEM_SHARED((N, MW), f32),
            pltpu.SemaphoreType.DMA,
            pltpu.SemaphoreType.DMA,
            pltpu.SemaphoreType.DMA,
            pltpu.SemaphoreType.DMA,
        ],
    )
    def gat_kernel(row_h, col_h, ea_h, lin_h, sij_h, ce_h, num_h,
                   idxr_all, idxc_all, eav0, eav1,
                   sr0, sr1, scb0, scb1, linb0, linb1,
                   mrg0, mrg1, cev, shacc,
                   sem_in0, sem_in1, sem_out0, sem_out1):
        c = lax.axis_index("c")
        s = lax.axis_index("s")
        io = lax.iota(jnp.int32, 16)
        zf = jnp.zeros((16,), f32)

        slots = ((sr0, scb0, linb0, mrg0, eav0, sem_in0, sem_out0),
                 (sr1, scb1, linb1, mrg1, eav1, sem_in1, sem_out1))

        pltpu.sync_copy(row_h.at[s], idxr_all)
        pltpu.sync_copy(col_h.at[s], idxc_all)
        pltpu.sync_copy(ce_h, cev)

        @pl.loop(0, K)
        def _zero(r):
            for kk in range(MW // 16):
                mrg0[r, pl.ds(kk * 16, 16)] = zf
                mrg1[r, pl.ds(kk * 16, 16)] = zf

        rb = pl.multiple_of(s * ROWS_PT, 8)
        for t in range(ROWS_PT // K):
            pltpu.sync_copy(mrg0, shacc.at[pl.ds(rb + t * K, K)])
        rem = ROWS_PT % K
        if rem:
            pltpu.sync_copy(mrg0.at[pl.ds(0, rem)],
                            shacc.at[pl.ds(rb + (ROWS_PT // K) * K, rem)])
        tail = N - NS * ROWS_PT

        @pl.when(s == NS - 1)
        def _zero_tail():
            pltpu.sync_copy(mrg0.at[pl.ds(0, tail)],
                            shacc.at[pl.ds(NS * ROWS_PT, tail)])

        plsc.subcore_barrier()

        cb4 = c * 4
        hsel = io & 3
        qsel = io >> 2
        cv = plsc.load_gather(cev, [cb4 + hsel])

        def in_descs(q, slot):
            sr, scb, linb, _, eav, sem_in, _ = slot
            ir = idxr_all.at[q]
            ic = idxc_all.at[q]
            return (pltpu.make_async_copy(sij_h.at[c].at[ir], sr, sem_in),
                    pltpu.make_async_copy(sij_h.at[c].at[ic], scb, sem_in),
                    pltpu.make_async_copy(lin_h.at[c].at[ic], linb, sem_in),
                    pltpu.make_async_copy(ea_h.at[s].at[q], eav, sem_in))

        def start_out(q, slot):
            mrg, sem_out = slot[3], slot[6]
            pltpu.async_copy(mrg, shacc.at[idxr_all.at[q]], sem_out, add=True)

        def wait_out(q, slot):
            mrg, sem_out = slot[3], slot[6]
            pltpu.make_async_copy(mrg, shacc.at[idxr_all.at[q]], sem_out).wait()

        def compute(q, slot):
            sr, scb, linb, mrg, eav, _, _ = slot
            if multi_head:
                @plsc.parallel_loop(0, K // 4, unroll=4)
                def _quad(i):
                    e4 = i * 4
                    rsel = e4 + qsel
                    si_v = plsc.load_gather(sr, [rsel, hsel])
                    sj_v = plsc.load_gather(scb, [rsel, hsel + 4])
                    ea_v = plsc.load_gather(eav, [rsel])
                    al = si_v + sj_v + ea_v * cv
                    al = jnp.maximum(al, 0.2 * al)
                    ex = jnp.exp(al)
                    plsc.store_scatter(mrg, [rsel, DH + cb4 + hsel], ex)
                    for l in range(4):
                        for g2 in range(2):
                            u = linb[e4 + l, pl.ds(g2 * 32, 32)]
                            av, bv = plsc.unpack(
                                u, format=plsc.PackFormat.INTERLEAVED,
                                preferred_element_type=f32)
                            mrg[e4 + l, pl.ds(g2 * 32, 16)] = (
                                av * _bcast_lane(ex, l * 4 + 2 * g2))
                            mrg[e4 + l, pl.ds(g2 * 32 + 16, 16)] = (
                                bv * _bcast_lane(ex, l * 4 + 2 * g2 + 1))
            else:
                io0 = io * 0

                @plsc.parallel_loop(0, K // 16, unroll=5)
                def _grp(j):
                    e0 = j * 16
                    rows = e0 + io
                    si_v = plsc.load_gather(sr, [rows, io0])
                    sj_v = plsc.load_gather(scb, [rows, io0 + 4])
                    ea_v = eav[pl.ds(e0, 16)]
                    al = si_v + sj_v + ea_v * cv
                    al = jnp.maximum(al, 0.2 * al)
                    ex = jnp.exp(al)
                    plsc.store_scatter(mrg, [rows, io0 + DH], ex)
                    for l in range(16):
                        m = _bcast_lane(ex, l)
                        for g2 in range(2):
                            u = linb[e0 + l, pl.ds(g2 * 32, 32)]
                            av, bv = plsc.unpack(
                                u, format=plsc.PackFormat.INTERLEAVED,
                                preferred_element_type=f32)
                            mrg[e0 + l, pl.ds(g2 * 32, 16)] = av * m
                            mrg[e0 + l, pl.ds(g2 * 32 + 16, 16)] = bv * m

        for d in in_descs(0, slots[0]):
            d.start()
        for d in in_descs(1, slots[1]):
            d.start()

        @pl.loop(0, NCHUNK // 2)
        def _body(t):
            for j in range(2):
                q = t * 2 + j
                slot = slots[j]
                for d in in_descs(q, slot):
                    d.wait()

                @pl.when(t > 0)
                def _drain():
                    wait_out(q, slot)

                compute(q, slot)
                start_out(q, slot)

                @pl.when(q + 2 < NCHUNK)
                def _refill():
                    for d in in_descs(q + 2, slot):
                        d.start()

        wait_out(0, slots[0])
        wait_out(1, slots[1])

        plsc.subcore_barrier()
        pltpu.sync_copy(shacc.at[pl.ds(rb, ROWS_PT)],
                        num_h.at[c].at[pl.ds(rb, ROWS_PT)])

        @pl.when(s == NS - 1)
        def _copy_tail():
            pltpu.sync_copy(shacc.at[pl.ds(NS * ROWS_PT, tail)],
                            num_h.at[c].at[pl.ds(NS * ROWS_PT, tail)])

    return gat_kernel(row3, col3, ea3, lin_hs, sij_hs, ce16)


BN = 1000
GRID = N // BN

def _blk(cols):
    return pl.BlockSpec((BN, cols), lambda i: (i, 0))

def _blk3(cols):
    return pl.BlockSpec((NC, BN, cols), lambda i: (0, i, 0))

def _full(r, cols):
    return pl.BlockSpec((r, cols), lambda i: (0, 0))


def _dot(a, b):
    return jnp.dot(a, b, preferred_element_type=f32)


def _lin_sij(xn, W, b, gl, ai, aj, SaA, SbA, SaB, SbB, P0, P1, lin_r, sij_r):
    t = _dot(xn, W[...]) + b[...]
    ln = _dot(t, gl[...])
    lin_r[0] = _dot(ln, P0[...]).astype(jnp.bfloat16)
    lin_r[1] = _dot(ln, P1[...]).astype(jnp.bfloat16)
    p = ln * ai[...]
    q = ln * aj[...]
    sij_r[0] = _dot(p, SaA[...]) + _dot(q, SbA[...])
    sij_r[1] = _dot(p, SaB[...]) + _dot(q, SbB[...])


def _tc1_body(x_r, W_r, b_r, gl_r, ai_r, aj_r, SaA_r, SbA_r, SaB_r, SbB_r,
              P0_r, P1_r, lin_r, sij_r):
    _lin_sij(x_r[...], W_r, b_r, gl_r, ai_r, aj_r, SaA_r, SbA_r, SaB_r, SbB_r,
             P0_r, P1_r, lin_r, sij_r)


def _tc1(x, W, b, gl, ai, aj, SaA, SbA, SaB, SbB, P0, P1):
    return pl.pallas_call(
        _tc1_body,
        grid=(GRID,),
        in_specs=[_blk(D), _full(D, D), _full(1, D), _full(D, D),
                  _full(1, D), _full(1, D),
                  _full(D, 8), _full(D, 8), _full(D, 8), _full(D, 8),
                  _full(D, DH), _full(D, DH)],
        out_specs=[_blk3(DH), _blk3(8)],
        out_shape=[jax.ShapeDtypeStruct((NC, N, DH), jnp.bfloat16),
                   jax.ShapeDtypeStruct((NC, N, 8), f32)],
    )(x, W, b, gl, ai, aj, SaA, SbA, SaB, SbB, P0, P1)


def _gat_combine(n_r, EXP_r):
    n0 = n_r[0]
    n1 = n_r[1]
    num = jnp.concatenate([n0[:, :DH], n1[:, :DH]], axis=-1)
    den = _dot(n0[:, DH:] + n1[:, DH:], EXP_r[...])
    return num / (den + 1e-16)


def _tcmid_body(n_r, xp_r, bng_r, bnb_r, Ws_r, bs_r, EXP_r,
                W_r, b_r, gl_r, ai_r, aj_r, SaA_r, SbA_r, SaB_r, SbB_r,
                P0_r, P1_r, xn_r, lin_r, sij_r):
    gat = _gat_combine(n_r, EXP_r)
    y = gat * (bng_r[...] * (1.0 / jnp.sqrt(1.0 + 1e-5))) + bnb_r[...]
    y = jnp.maximum(y, 0.0)
    xn = y + _dot(xp_r[...], Ws_r[...]) + bs_r[...]
    xn_r[...] = xn
    _lin_sij(xn, W_r, b_r, gl_r, ai_r, aj_r, SaA_r, SbA_r, SaB_r, SbB_r,
             P0_r, P1_r, lin_r, sij_r)


def _tcmid(num, xp, bng, bnb, Ws, bs, EXPd,
           W, b, gl, ai, aj, SaA, SbA, SaB, SbB, P0, P1):
    return pl.pallas_call(
        _tcmid_body,
        grid=(GRID,),
        in_specs=[_blk3(MW), _blk(D),
                  _full(1, D), _full(1, D), _full(D, D), _full(1, D),
                  _full(16, D),
                  _full(D, D), _full(1, D), _full(D, D),
                  _full(1, D), _full(1, D),
                  _full(D, 8), _full(D, 8), _full(D, 8), _full(D, 8),
                  _full(D, DH), _full(D, DH)],
        out_specs=[_blk(D), _blk3(DH), _blk3(8)],
        out_shape=[jax.ShapeDtypeStruct((N, D), f32),
                   jax.ShapeDtypeStruct((NC, N, DH), jnp.bfloat16),
                   jax.ShapeDtypeStruct((NC, N, 8), f32)],
    )(num, xp, bng, bnb, Ws, bs, EXPd, W, b, gl, ai, aj,
      SaA, SbA, SaB, SbB, P0, P1)


def _tcpost_body(n_r, xp_r, lng_r, lnb_r, Ws_r, bs_r, EXP_r, out_r):
    gat = _gat_combine(n_r, EXP_r)
    m = jnp.mean(gat, axis=-1, keepdims=True)
    v = jnp.mean((gat - m) ** 2, axis=-1, keepdims=True)
    y = (gat - m) / jnp.sqrt(v + 1e-5) * lng_r[...] + lnb_r[...]
    y = jnp.maximum(y, 0.0)
    out_r[...] = y + _dot(xp_r[...], Ws_r[...]) + bs_r[...]


def _tcpost(num, xp, lng, lnb, Ws, bs, EXPd):
    return pl.pallas_call(
        _tcpost_body,
        grid=(GRID,),
        in_specs=[_blk3(MW), _blk(D),
                  _full(1, D), _full(1, D), _full(D, D), _full(1, D),
                  _full(16, D)],
        out_specs=_blk(D),
        out_shape=jax.ShapeDtypeStruct((N, D), f32),
    )(num, xp, lng, lnb, Ws, bs, EXPd)


def kernel(x, edge_index, edge_attr, g1_lin, g1_edge, g1_att, g2_lin, g2_edge,
           g2_att, g3_lin, g3_edge, g3_att, W1, b1, W2, b2, W3, b3, Ws1, bs1,
           Ws2, bs2, Ws3, bs3, bn1_g, bn1_b, bn2_g, bn2_b, ln_g, ln_b):
    row3 = edge_index[0].reshape(NS, NCHUNK, K)
    col3 = edge_index[1].reshape(NS, NCHUNK, K)
    ea3 = edge_attr.reshape(NS, NCHUNK, K)

    r1 = lambda v: v.reshape(1, D)
    hh = jnp.arange(D, dtype=jnp.int32) // OC
    z4 = jnp.zeros((D, 4), f32)
    oh = lambda idx: jax.nn.one_hot(idx, 4, dtype=f32)
    SaA = jnp.concatenate([oh(hh), z4], axis=1)
    SbA = jnp.concatenate([z4, oh(hh)], axis=1)
    SaB = jnp.concatenate([oh(hh - 4), z4], axis=1)
    SbB = jnp.concatenate([z4, oh(hh - 4)], axis=1)
    EXPd = jax.nn.one_hot(hh, 16, dtype=f32).T
    jj = jnp.arange(DH, dtype=jnp.int32)
    gg = jj // 32
    pp = jj % 32
    def _perm(cc):
        srcc = cc * DH + gg * 32 + (pp % 2) * 16 + pp // 2
        return jax.nn.one_hot(srcc, D, dtype=f32).T
    P0 = _perm(0)
    P1 = _perm(1)

    ai1 = g1_att[0, :, :OC].reshape(1, D)
    aj1 = g1_att[0, :, OC:2 * OC].reshape(1, D)
    ce1 = (g1_edge.reshape(H, OC) * g1_att[0, :, 2 * OC:]).sum(-1)
    ce16_1 = jnp.tile(ce1, 2)
    ai2 = g2_att[0, :, :OC].reshape(1, D)
    aj2 = g2_att[0, :, OC:2 * OC].reshape(1, D)
    ce2 = (g2_edge.reshape(H, OC) * g2_att[0, :, 2 * OC:]).sum(-1)
    ce16_2 = jnp.tile(ce2, 2)

    ai3 = g3_att[0, 0, :D].reshape(1, D)
    aj3 = g3_att[0, 0, D:2 * D].reshape(1, D)
    ce3 = (g3_edge[0] * g3_att[0, 0, 2 * D:]).sum()
    ce16_3 = jnp.full((16,), ce3, f32)
    z128 = jnp.zeros((D,), jnp.int32)
    Sa3 = jax.nn.one_hot(z128, 8, dtype=f32)
    Sb3 = jax.nn.one_hot(z128 + 4, 8, dtype=f32)
    EXP3 = 0.5 * jax.nn.one_hot(z128, 16, dtype=f32).T

    lin1, sij1 = _tc1(x, W1, r1(b1), g1_lin, ai1, aj1, SaA, SbA, SaB, SbB, P0, P1)
    num1 = _sc_gat(row3, col3, ea3, lin1, sij1, ce16_1, True)
    x1, lin2, sij2 = _tcmid(num1, x,
                            r1(bn1_g), r1(bn1_b), Ws1, r1(bs1), EXPd,
                            W2, r1(b2), g2_lin, ai2, aj2, SaA, SbA, SaB, SbB, P0, P1)
    num2 = _sc_gat(row3, col3, ea3, lin2, sij2, ce16_2, True)
    x2, lin3, sij3 = _tcmid(num2, x1,
                            r1(bn2_g), r1(bn2_b), Ws2, r1(bs2), EXPd,
                            W3, r1(b3), g3_lin, ai3, aj3, Sa3, Sb3, Sa3, Sb3, P0, P1)
    num3 = _sc_gat(row3, col3, ea3, lin3, sij3, ce16_3, False)
    out = _tcpost(num3, x2, r1(ln_g), r1(ln_b), Ws3, r1(bs3), EXP3)
    return out
